# Initial kernel scaffold; baseline (speedup 1.0000x reference)
import jax
import jax.numpy as jnp
from jax.experimental import pallas as pl


def kernel(x_cell, edge_index_line, edge_index_region, edge_index_diag, W_gat, att_src, att_dst, b_gat, bn_gamma, bn_beta, Wp, bp, Wk, bk, Wq, bq, Wv, bv, a_rel, m_rel, p_rel, Wo, bo, skip, gf_gamma, gf_beta, Wl, bl):
    raise NotImplementedError("write your pallas kernel here")


# speedup vs baseline: 39.5290x; 39.5290x over previous
import functools

import jax
import jax.numpy as jnp
from jax import lax
from jax.experimental import pallas as pl
from jax.experimental.pallas import tpu as pltpu
from jax.experimental.pallas import tpu_sc as plsc

N = 10000
E = 200000
H = 2
D = 64
HID = 128
NT = 3
NL = 3

NC = 2
NS = 16
NW = NC * NS
CK = 128
DW = 8

MG = 3 * N + 208
MO = N + 240

LG = NT * (E + N)
LH = NT * E


def _pad_edges(L):
    ch = -(-L // (NW * CK))
    return ch * NW * CK, ch


PG, CHG = _pad_edges(LG)
PH, CHH = _pad_edges(LH)

_MESH = plsc.VectorSubcoreMesh(
    core_axis_name="c", subcore_axis_name="s", num_cores=NC, num_subcores=NS)

BS = 1000


def _sc_gat_logits(CH):
    @functools.partial(
        pl.kernel, mesh=_MESH,
        compiler_params=pltpu.CompilerParams(needs_layout_passes=False),
        out_type=(jax.ShapeDtypeStruct((NC * 2 * MG,), jnp.float32),
                  jax.ShapeDtypeStruct((2 * PG,), jnp.float32)),
        scratch_types=[
            pltpu.VMEM((1, CK), jnp.int32),
            pltpu.VMEM((1, CK), jnp.int32),
            pltpu.VMEM((1, CK), jnp.int32),
            pltpu.VMEM((CK,), jnp.float32),
            pltpu.VMEM((CK,), jnp.float32),
            pltpu.VMEM((CK,), jnp.float32),
            pltpu.VMEM((CK,), jnp.float32),
            pltpu.VMEM((CK,), jnp.float32),
            pltpu.VMEM((CK,), jnp.float32),
            pltpu.VMEM((MG // NS,), jnp.float32),
            pltpu.VMEM_SHARED((MG,), jnp.float32),
            pltpu.VMEM_SHARED((MG,), jnp.float32),
            pltpu.SemaphoreType.DMA,
        ],
    )
    def body(sg, dpg, asrcf, adstf, den_out, ex_out,
             sbuf, dbuf, ib2, exb0, exb1, ab0, ab1, bb0, bb1, zbuf,
             den0_sh, den1_sh, sem):
        c = lax.axis_index("c")
        s = lax.axis_index("s")
        w = c * NS + s
        rp = MG // NS
        z16f = jnp.zeros((16,), jnp.float32)

        def zloop(j, cc):
            zbuf[pl.ds(16 * j, 16)] = z16f
            return cc

        lax.fori_loop(0, rp // 16, zloop, 0)
        pltpu.sync_copy(zbuf, den0_sh.at[pl.ds(s * rp, rp)])
        pltpu.sync_copy(zbuf, den1_sh.at[pl.ds(s * rp, rp)])
        plsc.subcore_barrier()

        def chunk(k, carry):
            row = w * CH + k
            base = row * CK
            pltpu.sync_copy(sg.at[row], sbuf)
            pltpu.sync_copy(dpg.at[row], dbuf)
            pltpu.async_copy(asrcf.at[sbuf.at[0]], ab0, sem).wait()
            pltpu.async_copy(adstf.at[dbuf.at[0]], bb0, sem).wait()
            for t in range(CK // 16):
                sl = pl.ds(16 * t, 16)
                ib2[0, sl] = sbuf[0, sl] + MG
            pltpu.async_copy(asrcf.at[ib2.at[0]], ab1, sem).wait()
            for t in range(CK // 16):
                sl = pl.ds(16 * t, 16)
                ib2[0, sl] = dbuf[0, sl] + MG
            pltpu.async_copy(adstf.at[ib2.at[0]], bb1, sem).wait()
            for t in range(CK // 16):
                sl = pl.ds(16 * t, 16)
                for exb, aa, bb in ((exb0, ab0, bb0), (exb1, ab1, bb1)):
                    al = aa[sl] + bb[sl]
                    al = jnp.where(al >= 0.0, al, 0.2 * al)
                    exb[sl] = jnp.exp(al)
            pltpu.sync_copy(exb0, den0_sh.at[dbuf.at[0]], add=True)
            pltpu.sync_copy(exb1, den1_sh.at[dbuf.at[0]], add=True)
            pltpu.sync_copy(exb0, ex_out.at[pl.ds(base, CK)])
            pltpu.sync_copy(exb1, ex_out.at[pl.ds(PG + base, CK)])
            return carry

        lax.fori_loop(0, CH, chunk, 0)
        plsc.subcore_barrier()
        pltpu.sync_copy(den0_sh.at[pl.ds(s * rp, rp)], zbuf)
        pltpu.sync_copy(zbuf, den_out.at[pl.ds((c * 2) * MG + s * rp, rp)])
        pltpu.sync_copy(den1_sh.at[pl.ds(s * rp, rp)], zbuf)
        pltpu.sync_copy(zbuf, den_out.at[pl.ds((c * 2 + 1) * MG + s * rp, rp)])

    return body


def _sc_hgt_logits(CH):
    @functools.partial(
        pl.kernel, mesh=_MESH,
        compiler_params=pltpu.CompilerParams(needs_layout_passes=False),
        out_type=(jax.ShapeDtypeStruct((NC * 2 * MO,), jnp.float32),
                  jax.ShapeDtypeStruct((2 * PH,), jnp.float32)),
        scratch_types=[
            pltpu.VMEM((1, CK), jnp.int32),
            pltpu.VMEM((1, CK), jnp.int32),
            pltpu.VMEM((CK, HID), jnp.float32),
            pltpu.VMEM((CK, HID), jnp.float32),
            pltpu.VMEM((CK,), jnp.float32),
            pltpu.VMEM((CK,), jnp.float32),
            pltpu.VMEM((MO // NS,), jnp.float32),
            pltpu.VMEM_SHARED((MO,), jnp.float32),
            pltpu.VMEM_SHARED((MO,), jnp.float32),
            pltpu.SemaphoreType.DMA,
        ],
    )
    def body(sh, dh, qtab, ktab, den_out, ex_out,
             sbuf, dbuf, qrows, krows, exb0, exb1, zbuf,
             den0_sh, den1_sh, sem):
        c = lax.axis_index("c")
        s = lax.axis_index("s")
        w = c * NS + s
        rp = MO // NS
        z16f = jnp.zeros((16,), jnp.float32)

        def zloop(j, cc):
            zbuf[pl.ds(16 * j, 16)] = z16f
            return cc

        lax.fori_loop(0, rp // 16, zloop, 0)
        pltpu.sync_copy(zbuf, den0_sh.at[pl.ds(s * rp, rp)])
        pltpu.sync_copy(zbuf, den1_sh.at[pl.ds(s * rp, rp)])
        plsc.subcore_barrier()
        iot = lax.iota(jnp.int32, 16)

        def chunk(k, carry):
            row = w * CH + k
            base = row * CK
            pltpu.sync_copy(sh.at[row], sbuf)
            pltpu.sync_copy(dh.at[row], dbuf)
            pltpu.async_copy(ktab.at[sbuf.at[0]], krows, sem).wait()
            pltpu.async_copy(qtab.at[dbuf.at[0]], qrows, sem).wait()

            def dotg(m, cc):
                acc0 = jnp.zeros((16,), jnp.float32)
                acc1 = jnp.zeros((16,), jnp.float32)
                for e in range(16):
                    j = 16 * m + e
                    for h in range(H):
                        bcol = h * D
                        acc = (qrows[j, pl.ds(bcol, 16)]
                               * krows[j, pl.ds(bcol, 16)])
                        for gg in range(1, D // 16):
                            acc = acc + (
                                qrows[j, pl.ds(bcol + 16 * gg, 16)]
                                * krows[j, pl.ds(bcol + 16 * gg, 16)])
                        dot = jnp.sum(acc)
                        if h == 0:
                            acc0 = jnp.where(iot == e, dot, acc0)
                        else:
                            acc1 = jnp.where(iot == e, dot, acc1)
                exb0[pl.ds(16 * m, 16)] = jnp.exp(acc0)
                exb1[pl.ds(16 * m, 16)] = jnp.exp(acc1)
                return cc

            lax.fori_loop(0, CK // 16, dotg, 0)
            pltpu.sync_copy(exb0, den0_sh.at[dbuf.at[0]], add=True)
            pltpu.sync_copy(exb1, den1_sh.at[dbuf.at[0]], add=True)
            pltpu.sync_copy(exb0, ex_out.at[pl.ds(base, CK)])
            pltpu.sync_copy(exb1, ex_out.at[pl.ds(PH + base, CK)])
            return carry

        lax.fori_loop(0, CH, chunk, 0)
        plsc.subcore_barrier()
        pltpu.sync_copy(den0_sh.at[pl.ds(s * rp, rp)], zbuf)
        pltpu.sync_copy(zbuf, den_out.at[pl.ds((c * 2) * MO + s * rp, rp)])
        pltpu.sync_copy(den1_sh.at[pl.ds(s * rp, rp)], zbuf)
        pltpu.sync_copy(zbuf, den_out.at[pl.ds((c * 2 + 1) * MO + s * rp, rp)])

    return body


def _sc_aggregate(CH, MI, P):
    @functools.partial(
        pl.kernel, mesh=_MESH,
        compiler_params=pltpu.CompilerParams(needs_layout_passes=False),
        out_type=jax.ShapeDtypeStruct((NC, MO, HID), jnp.float32),
        scratch_types=[
            pltpu.VMEM((1, CK), jnp.int32),
            pltpu.VMEM((1, CK), jnp.int32),
            pltpu.VMEM((1, CK), jnp.int32),
            pltpu.VMEM((1, CK), jnp.int32),
            pltpu.VMEM((CK,), jnp.float32),
            pltpu.VMEM((CK,), jnp.float32),
            pltpu.VMEM((CK,), jnp.float32),
            pltpu.VMEM((CK,), jnp.float32),
            pltpu.VMEM((CK, HID), jnp.float32),
            pltpu.VMEM_SHARED((MO, HID), jnp.float32),
            pltpu.SemaphoreType.DMA,
        ],
    )
    def body(sg, pg, dg, invd, exa, table, out_pair,
             sbuf, pbuf, dbuf, ib2, wbuf0, wbuf1, ibuf0, ibuf1,
             rows, out_sh, sem):
        c = lax.axis_index("c")
        s = lax.axis_index("s")
        w = c * NS + s
        rp = MO // NS
        z16f = jnp.zeros((16,), jnp.float32)

        def zrow_loop(j, cc):
            for g in range(HID // 16):
                rows[j, pl.ds(16 * g, 16)] = z16f
            return cc

        lax.fori_loop(0, CK, zrow_loop, 0)

        def zcp(i, cc):
            pltpu.sync_copy(rows, out_sh.at[pl.ds(s * rp + i * CK, CK)])
            return cc

        lax.fori_loop(0, rp // CK, zcp, 0)
        plsc.subcore_barrier()

        def chunk(k, carry):
            row = w * CH + k
            base = row * CK
            pltpu.sync_copy(sg.at[row], sbuf)
            pltpu.sync_copy(pg.at[row], pbuf)
            pltpu.sync_copy(dg.at[row], dbuf)
            pltpu.sync_copy(exa.at[pl.ds(base, CK)], wbuf0)
            pltpu.sync_copy(exa.at[pl.ds(P + base, CK)], wbuf1)
            pltpu.async_copy(table.at[sbuf.at[0]], rows, sem).wait()
            pltpu.async_copy(invd.at[pbuf.at[0]], ibuf0, sem).wait()
            for t in range(CK // 16):
                sl = pl.ds(16 * t, 16)
                ib2[0, sl] = pbuf[0, sl] + MI
            pltpu.async_copy(invd.at[ib2.at[0]], ibuf1, sem).wait()
            for t in range(CK // 16):
                sl = pl.ds(16 * t, 16)
                wbuf0[sl] = wbuf0[sl] * ibuf0[sl]
                wbuf1[sl] = wbuf1[sl] * ibuf1[sl]

            def scale(m, cc):
                wv0 = wbuf0[pl.ds(16 * m, 16)]
                wv1 = wbuf1[pl.ds(16 * m, 16)]
                for e in range(16):
                    j = 16 * m + e
                    w0 = wv0[e]
                    w1 = wv1[e]
                    for g in range(HID // 16):
                        wg = w0 if g < (HID // 32) else w1
                        rows[j, pl.ds(16 * g, 16)] = (
                            rows[j, pl.ds(16 * g, 16)] * wg)
                return cc

            lax.fori_loop(0, CK // 16, scale, 0)
            pltpu.sync_copy(rows, out_sh.at[dbuf.at[0]], add=True)
            return carry

        lax.fori_loop(0, CH, chunk, 0)
        plsc.subcore_barrier()

        def wout(i, cc):
            pltpu.sync_copy(out_sh.at[pl.ds(s * rp + i * CK, CK)], rows)
            pltpu.sync_copy(rows, out_pair.at[c, pl.ds(s * rp + i * CK, CK)])
            return cc

        lax.fori_loop(0, rp // CK, wout, 0)

    return body


_GAT_LOGITS = _sc_gat_logits(CHG)
_HGT_LOGITS = _sc_hgt_logits(CHH)
_GAT_AGG = _sc_aggregate(CHG, MG, PG)
_HGT_AGG = _sc_aggregate(CHH, MO, PH)


def _gat_node(h, W3, As, Ad):
    def f(h_ref, w_ref, as_ref, ad_ref, xw_ref, s_ref, d_ref):
        hb = h_ref[...]
        for t in range(NT):
            xwt = jnp.dot(hb, w_ref[t], preferred_element_type=jnp.float32)
            xw_ref[t] = xwt
            s_ref[t] = jnp.dot(xwt, as_ref[t],
                               preferred_element_type=jnp.float32)
            d_ref[t] = jnp.dot(xwt, ad_ref[t],
                               preferred_element_type=jnp.float32)

    return pl.pallas_call(
        f, grid=(N // BS,),
        in_specs=[pl.BlockSpec((BS, HID), lambda i: (i, 0)),
                  pl.BlockSpec((NT, HID, HID), lambda i: (0, 0, 0)),
                  pl.BlockSpec((NT, HID, 2), lambda i: (0, 0, 0)),
                  pl.BlockSpec((NT, HID, 2), lambda i: (0, 0, 0))],
        out_specs=[pl.BlockSpec((NT, BS, HID), lambda i: (0, i, 0)),
                   pl.BlockSpec((NT, BS, 2), lambda i: (0, i, 0)),
                   pl.BlockSpec((NT, BS, 2), lambda i: (0, i, 0))],
        out_shape=[jax.ShapeDtypeStruct((NT, N, HID), jnp.float32),
                   jax.ShapeDtypeStruct((NT, N, 2), jnp.float32),
                   jax.ShapeDtypeStruct((NT, N, 2), jnp.float32)],
    )(h, W3, As, Ad)


def _px_pair(x0, Wp, bp2):
    def f(x_ref, w_ref, b_ref, o_ref):
        xb = x_ref[...]
        for l in range(2):
            o_ref[l] = jnp.dot(xb, w_ref[l],
                               preferred_element_type=jnp.float32) + b_ref[l]

    return pl.pallas_call(
        f, grid=(N // BS,),
        in_specs=[pl.BlockSpec((BS, HID), lambda i: (i, 0)),
                  pl.BlockSpec((2, HID, HID), lambda i: (0, 0, 0)),
                  pl.BlockSpec((2, 1, HID), lambda i: (0, 0, 0))],
        out_specs=pl.BlockSpec((2, BS, HID), lambda i: (0, i, 0)),
        out_shape=jax.ShapeDtypeStruct((2, N, HID), jnp.float32),
    )(x0, Wp, bp2)


def _invden(den_pair):
    M = den_pair.shape[2]

    def f(d_ref, o_ref):
        o_ref[...] = 1.0 / (d_ref[0] + d_ref[1] + 1e-16)

    return pl.pallas_call(
        f, out_shape=jax.ShapeDtypeStruct((2, M), jnp.float32),
    )(den_pair)


def _gsum(out_pair):
    def f(p_ref, g_ref):
        g_ref[...] = p_ref[0] + p_ref[1]

    return pl.pallas_call(
        f, grid=(N // BS,),
        in_specs=[pl.BlockSpec((2, BS, HID), lambda i: (0, i, 0))],
        out_specs=pl.BlockSpec((BS, HID), lambda i: (i, 0)),
        out_shape=jax.ShapeDtypeStruct((N, HID), jnp.float32),
    )(out_pair)


def _stats(g):
    def f(g_ref, o_ref):
        gv = g_ref[...]
        m = jnp.mean(gv, axis=0)
        v = jnp.mean((gv - m[None, :]) ** 2, axis=0)
        o_ref[0] = m
        o_ref[1] = v

    return pl.pallas_call(
        f, out_shape=jax.ShapeDtypeStruct((2, HID), jnp.float32),
    )(g)


def _apply(hprev, g, px, st, gamma, beta):
    def f(h_ref, g_ref, p_ref, s_ref, gm_ref, bt_ref, o_ref):
        m = s_ref[0]
        v = s_ref[1]
        bn = (g_ref[...] - m) * lax.rsqrt(v + 1e-5) * gm_ref[0] + bt_ref[0]
        t = h_ref[...] + bn + p_ref[...]
        o_ref[...] = jnp.where(t >= 0.0, t, 0.2 * t)

    return pl.pallas_call(
        f, grid=(N // BS,),
        in_specs=[pl.BlockSpec((BS, HID), lambda i: (i, 0)),
                  pl.BlockSpec((BS, HID), lambda i: (i, 0)),
                  pl.BlockSpec((BS, HID), lambda i: (i, 0)),
                  pl.BlockSpec((2, HID), lambda i: (0, 0)),
                  pl.BlockSpec((1, HID), lambda i: (0, 0)),
                  pl.BlockSpec((1, HID), lambda i: (0, 0))],
        out_specs=pl.BlockSpec((BS, HID), lambda i: (i, 0)),
        out_shape=jax.ShapeDtypeStruct((N, HID), jnp.float32),
    )(hprev, g, px, st, gamma, beta)


def _hgt_node(h, Wk, bk2, Wq, bq2, Wv, bv2, AK, AM):
    def f(h_ref, wk_ref, bk_ref, wq_ref, bq_ref, wv_ref, bv_ref,
          ak_ref, am_ref, q_ref, kes_ref, ves_ref):
        hb = h_ref[...]
        kb = jnp.dot(hb, wk_ref[...],
                     preferred_element_type=jnp.float32) + bk_ref[0]
        qb = jnp.dot(hb, wq_ref[...],
                     preferred_element_type=jnp.float32) + bq_ref[0]
        vb = jnp.dot(hb, wv_ref[...],
                     preferred_element_type=jnp.float32) + bv_ref[0]
        q_ref[...] = qb
        for t in range(NT):
            kes_ref[t] = jnp.dot(kb, ak_ref[t],
                                 preferred_element_type=jnp.float32)
            ves_ref[t] = jnp.dot(vb, am_ref[t],
                                 preferred_element_type=jnp.float32)

    full = lambda i: (0, 0)
    full3 = lambda i: (0, 0, 0)
    return pl.pallas_call(
        f, grid=(N // BS,),
        in_specs=[pl.BlockSpec((BS, HID), lambda i: (i, 0)),
                  pl.BlockSpec((HID, HID), full),
                  pl.BlockSpec((1, HID), full),
                  pl.BlockSpec((HID, HID), full),
                  pl.BlockSpec((1, HID), full),
                  pl.BlockSpec((HID, HID), full),
                  pl.BlockSpec((1, HID), full),
                  pl.BlockSpec((NT, HID, HID), full3),
                  pl.BlockSpec((NT, HID, HID), full3)],
        out_specs=[pl.BlockSpec((BS, HID), lambda i: (i, 0)),
                   pl.BlockSpec((NT, BS, HID), lambda i: (0, i, 0)),
                   pl.BlockSpec((NT, BS, HID), lambda i: (0, i, 0))],
        out_shape=[jax.ShapeDtypeStruct((N, HID), jnp.float32),
                   jax.ShapeDtypeStruct((NT, N, HID), jnp.float32),
                   jax.ShapeDtypeStruct((NT, N, HID), jnp.float32)],
    )(h, Wk, bk2, Wq, bq2, Wv, bv2, AK, AM)


def _hgt_pre(out_pair, h, Wo, s_arr):
    def f(p_ref, h_ref, wo_ref, s_ref, o_ref):
        g = p_ref[0] + p_ref[1]
        ge = 0.5 * g * (1.0 + lax.erf(g * 0.7071067811865476))
        out = jnp.dot(ge, wo_ref[...], preferred_element_type=jnp.float32)
        sv = s_ref[...]
        o_ref[...] = sv * out + (1.0 - sv) * h_ref[...]

    return pl.pallas_call(
        f, grid=(N // BS,),
        in_specs=[pl.BlockSpec((2, BS, HID), lambda i: (0, i, 0)),
                  pl.BlockSpec((BS, HID), lambda i: (i, 0)),
                  pl.BlockSpec((HID, HID), lambda i: (0, 0)),
                  pl.BlockSpec((1, HID), lambda i: (0, 0))],
        out_specs=pl.BlockSpec((BS, HID), lambda i: (i, 0)),
        out_shape=jax.ShapeDtypeStruct((N, HID), jnp.float32),
    )(out_pair, h, Wo, s_arr)


def _final(h2, Wl, bl2):
    def f(h_ref, w_ref, b_ref, o_ref):
        o_ref[...] = jnp.dot(h_ref[...], w_ref[...],
                             preferred_element_type=jnp.float32) + b_ref[...]

    return pl.pallas_call(
        f, grid=(N // BS,),
        in_specs=[pl.BlockSpec((BS, HID), lambda i: (i, 0)),
                  pl.BlockSpec((HID, 1), lambda i: (0, 0)),
                  pl.BlockSpec((1, 1), lambda i: (0, 0))],
        out_specs=pl.BlockSpec((BS, 1), lambda i: (i, 0)),
        out_shape=jax.ShapeDtypeStruct((N, 1), jnp.float32),
    )(h2, Wl, bl2)


def kernel(x_cell, edge_index_line, edge_index_region, edge_index_diag,
           W_gat, att_src, att_dst, b_gat, bn_gamma, bn_beta, Wp, bp,
           Wk, bk, Wq, bq, Wv, bv, a_rel, m_rel, p_rel, Wo, bo, skip,
           gf_gamma, gf_beta, Wl, bl):
    f32 = jnp.float32
    i32 = jnp.int32
    x0 = x_cell.astype(f32)
    edges = (edge_index_line, edge_index_region, edge_index_diag)

    ar = jnp.arange(N, dtype=i32)
    sg, dg, dpg = [], [], []
    for t, ei in enumerate(edges):
        s0 = jnp.concatenate([ei[0].astype(i32), ar])
        d0 = jnp.concatenate([ei[1].astype(i32), ar])
        sg.append(s0 + t * N)
        dg.append(d0)
        dpg.append(d0 + t * N)
    padg = PG - LG
    SG = jnp.concatenate(sg + [jnp.zeros((padg,), i32)]).reshape(-1, 1, CK)
    DG = jnp.concatenate(dg + [jnp.full((padg,), N, i32)]).reshape(-1, 1, CK)
    DPG = jnp.concatenate(
        dpg + [jnp.full((padg,), 3 * N, i32)]).reshape(-1, 1, CK)

    padh = PH - LH
    SH = jnp.concatenate([ei[0].astype(i32) + t * N
                          for t, ei in enumerate(edges)]
                         + [jnp.zeros((padh,), i32)])
    DH = jnp.concatenate([ei[1].astype(i32) for ei in edges]
                         + [jnp.full((padh,), N, i32)])
    SH = SH.reshape(-1, 1, CK)
    DH = DH.reshape(-1, 1, CK)

    zh = jnp.zeros((N, HID), f32)

    bp2 = bp.astype(f32).reshape(2, 1, HID)
    px12 = _px_pair(x0, Wp.astype(f32), bp2)

    def head_mat(a):
        m = jnp.zeros((NT, HID, 2), f32)
        m = m.at[:, 0:D, 0].set(a[:, 0])
        m = m.at[:, D:HID, 1].set(a[:, 1])
        return m

    def rel_mat(a, scale):
        m = jnp.zeros((NT, HID, HID), f32)
        m = m.at[:, 0:D, 0:D].set(a[:, 0] * scale[:, 0, None, None])
        m = m.at[:, D:HID, D:HID].set(a[:, 1] * scale[:, 1, None, None])
        return m

    h = None
    for l in range(NL):
        hin = x0 if l == 0 else h
        As = head_mat(att_src[l].astype(f32))
        Ad = head_mat(att_dst[l].astype(f32))
        xw3, s3, d3 = _gat_node(hin, W_gat[l].astype(f32), As, Ad)
        XW = xw3.reshape(NT * N, HID)
        asrcf = jnp.pad(s3.reshape(NT * N, 2).T, ((0, 0), (0, MG - NT * N))
                        ).reshape(-1)
        adstf = jnp.pad(d3.reshape(NT * N, 2).T, ((0, 0), (0, MG - NT * N))
                        ).reshape(-1)
        den_pair, exg = _GAT_LOGITS(SG, DPG, asrcf, adstf)
        invd = _invden(den_pair.reshape(NC, 2, MG)).reshape(-1)
        out_pair = _GAT_AGG(SG, DPG, DG, invd, exg, XW)
        g = _gsum(out_pair)
        st = _stats(g)
        hprev = zh if l == 0 else h
        pxl = zh if l == 0 else px12[l - 1]
        h = _apply(hprev, g, pxl, st,
                   bn_gamma[l].astype(f32).reshape(1, HID),
                   bn_beta[l].astype(f32).reshape(1, HID))

    scale = (p_rel.astype(f32) / jnp.sqrt(jnp.asarray(float(D), f32)))
    AK = rel_mat(a_rel.astype(f32), scale)
    AM = rel_mat(m_rel.astype(f32), jnp.ones_like(scale))
    q, kes, ves = _hgt_node(
        h, Wk.astype(f32), bk.astype(f32).reshape(1, HID),
        Wq.astype(f32), bq.astype(f32).reshape(1, HID),
        Wv.astype(f32), bv.astype(f32).reshape(1, HID), AK, AM)
    qpad = jnp.concatenate([q, jnp.zeros((MO - N, HID), f32)])
    KES = kes.reshape(NT * N, HID)
    VES = ves.reshape(NT * N, HID)
    den_pair_h, exh = _HGT_LOGITS(SH, DH, qpad, KES)
    invdh = _invden(den_pair_h.reshape(NC, 2, MO)).reshape(-1)
    out_pair_h = _HGT_AGG(SH, DH, DH, invdh, exh, VES)
    sv = jnp.broadcast_to(jax.nn.sigmoid(skip.astype(f32)), (1, HID))
    hn = _hgt_pre(out_pair_h, h, Wo.astype(f32), sv)
    st_h = _stats(hn)
    h2 = _apply(h, hn, zh, st_h,
                gf_gamma.astype(f32).reshape(1, HID),
                gf_beta.astype(f32).reshape(1, HID))
    y = _final(h2, Wl.astype(f32), bl.astype(f32).reshape(1, 1))
    return y[:, 0]
